# Initial kernel scaffold; baseline (speedup 1.0000x reference)
import jax
import jax.numpy as jnp
from jax.experimental import pallas as pl


def kernel(x, t, gamma1, beta1, W_red, gamma2, beta2, w_gate, We1, be1, We2, be2, modal_shifts, task_index):
    raise NotImplementedError("write your pallas kernel here")


# speedup vs baseline: 2.0530x; 2.0530x over previous
import jax
import jax.numpy as jnp
from jax.experimental import pallas as pl
from jax.experimental.pallas import tpu as pltpu

DIM = 1024
RED = 256
E = 4
HID = 32
EH = E * HID
TB = 256


def _body(x_ref, t_ref, wxt_ref, c2_ref, w1g_ref, b1g_ref,
          w2_ref, be2_ref, sx_ref, st_ref,
          ox_ref, ot_ref, px_ref, pt_ref, aux_ref,
          imp_ref, load_ref):
    i = pl.program_id(0)
    nsteps = pl.num_programs(0)

    xb = x_ref[...]
    tb = t_ref[...]

    m = (jnp.sum(xb, axis=1, keepdims=True)
         + jnp.sum(tb, axis=1, keepdims=True)) * (1.0 / (2 * DIM))
    xc = xb - m
    tc = tb - m
    v = (jnp.sum(xc * xc, axis=1, keepdims=True)
         + jnp.sum(tc * tc, axis=1, keepdims=True)) * (1.0 / (2 * DIM))
    s = jnp.sqrt(v + 1e-5)
    xn = xc / s
    tn = tc / s

    yn = jnp.concatenate([xn, tn], axis=1).astype(jnp.bfloat16)
    u = jnp.dot(yn, wxt_ref[...], preferred_element_type=jnp.float32)
    u = u + c2_ref[...]

    m2 = jnp.mean(u, axis=1, keepdims=True)
    uc = u - m2
    v2 = jnp.mean(uc * uc, axis=1, keepdims=True)
    z = uc / jnp.sqrt(v2 + 1e-5)

    zb = z.astype(jnp.bfloat16)
    r = jnp.dot(zb, w1g_ref[...], preferred_element_type=jnp.float32) + b1g_ref[...]
    h = jnp.maximum(r[:, :EH], 0.0)
    logits = r[:, EH:EH + E]

    iota = jax.lax.broadcasted_iota(jnp.int32, logits.shape, 1)
    m1 = jnp.max(logits, axis=1, keepdims=True)
    idx1 = jnp.min(jnp.where(logits == m1, iota, E), axis=1, keepdims=True)
    l2 = jnp.where(iota == idx1, -jnp.inf, logits)
    m2g = jnp.max(l2, axis=1, keepdims=True)
    idx2 = jnp.min(jnp.where(l2 == m2g, iota, E), axis=1, keepdims=True)
    e2 = jnp.exp(m2g - m1)
    den = 1.0 + e2
    gates = (jnp.where(iota == idx1, 1.0 / den, 0.0)
             + jnp.where(iota == idx2, e2 / den, 0.0))

    hb = h.astype(jnp.bfloat16)
    moe = jnp.zeros((h.shape[0], RED), jnp.float32)
    for e in range(E):
        exp_o = jnp.dot(hb[:, e * HID:(e + 1) * HID],
                        w2_ref[e * HID:(e + 1) * HID, :],
                        preferred_element_type=jnp.float32)
        moe = moe + gates[:, e:e + 1] * (exp_o + be2_ref[e:e + 1, :])

    px = jax.nn.sigmoid(jnp.mean(moe[:, :RED // 2], axis=1, keepdims=True))
    pt = jax.nn.sigmoid(jnp.mean(moe[:, RED // 2:], axis=1, keepdims=True))

    ox_ref[...] = px * xb + sx_ref[...]
    ot_ref[...] = pt * tb + st_ref[...]
    px_ref[...] = px
    pt_ref[...] = pt

    imp_b = jnp.sum(gates, axis=0, keepdims=True)
    load_b = jnp.sum((gates > 0.0).astype(jnp.float32), axis=0, keepdims=True)

    @pl.when(i == 0)
    def _init():
        imp_ref[...] = imp_b
        load_ref[...] = load_b

    @pl.when(i > 0)
    def _acc():
        imp_ref[...] += imp_b
        load_ref[...] += load_b

    @pl.when(i == nsteps - 1)
    def _fin():
        def cv2(a):
            mu = jnp.sum(a, axis=1, keepdims=True) * (1.0 / E)
            var = jnp.sum((a - mu) ** 2, axis=1, keepdims=True) * (1.0 / (E - 1))
            return var / (mu * mu + 1e-10)

        aux_ref[...] = (cv2(imp_ref[...]) + cv2(load_ref[...])) * 1e-2


def kernel(x, t, gamma1, beta1, W_red, gamma2, beta2, w_gate, We1, be1, We2,
           be2, modal_shifts, task_index):
    B, N, C = x.shape
    T = B * N
    xf = x.reshape(T, C)
    tf = t.reshape(T, C)

    Wxt = (W_red * gamma1[None, :]).T.astype(jnp.bfloat16)
    c2 = jnp.sum(W_red * beta1[None, :], axis=1)[None, :]
    wg = w_gate[task_index]
    We1f = We1.transpose(1, 0, 2).reshape(RED, EH)
    w1g = jnp.concatenate([We1f * gamma2[:, None], wg * gamma2[:, None]],
                          axis=1).astype(jnp.bfloat16)
    b1g = jnp.concatenate([beta2 @ We1f + be1.reshape(EH), beta2 @ wg])[None, :]
    W2 = We2.reshape(EH, RED).astype(jnp.bfloat16)
    sx = modal_shifts[task_index * 2 + 0][None, :]
    st = modal_shifts[task_index * 2 + 1][None, :]

    grid = (T // TB,)
    tok = lambda i: (i, 0)
    fix = lambda i: (0, 0)

    out_x, out_t, pxo, pto, aux = pl.pallas_call(
        _body,
        grid=grid,
        in_specs=[
            pl.BlockSpec((TB, C), tok),
            pl.BlockSpec((TB, C), tok),
            pl.BlockSpec((2 * C, RED), fix),
            pl.BlockSpec((1, RED), fix),
            pl.BlockSpec((RED, EH + E), fix),
            pl.BlockSpec((1, EH + E), fix),
            pl.BlockSpec((EH, RED), fix),
            pl.BlockSpec((E, RED), fix),
            pl.BlockSpec((1, C), fix),
            pl.BlockSpec((1, C), fix),
        ],
        out_specs=[
            pl.BlockSpec((TB, C), tok),
            pl.BlockSpec((TB, C), tok),
            pl.BlockSpec((TB, 1), tok),
            pl.BlockSpec((TB, 1), tok),
            pl.BlockSpec((1, 1), fix),
        ],
        out_shape=[
            jax.ShapeDtypeStruct((T, C), jnp.float32),
            jax.ShapeDtypeStruct((T, C), jnp.float32),
            jax.ShapeDtypeStruct((T, 1), jnp.float32),
            jax.ShapeDtypeStruct((T, 1), jnp.float32),
            jax.ShapeDtypeStruct((1, 1), jnp.float32),
        ],
        scratch_shapes=[
            pltpu.VMEM((1, E), jnp.float32),
            pltpu.VMEM((1, E), jnp.float32),
        ],
        compiler_params=pltpu.CompilerParams(
            dimension_semantics=("arbitrary",),
        ),
    )(xf, tf, Wxt, c2, w1g, b1g, W2, be2, sx, st)

    return (out_x.reshape(B, N, C), out_t.reshape(B, N, C),
            pxo.reshape(B, N, 1), pto.reshape(B, N, 1),
            aux.reshape(()))


# speedup vs baseline: 3.0025x; 1.4625x over previous
import jax
import jax.numpy as jnp
from jax.experimental import pallas as pl
from jax.experimental.pallas import tpu as pltpu

DIM = 1024
RED = 256
E = 4
HID = 32
EH = E * HID
TB = 512


def _body(x_ref, t_ref, wr_ref, g1_ref, b1_ref, g2_ref, b2_ref, wg_ref,
          we1_ref, be1_ref, we2_ref, be2_ref, ms_ref, ti_ref,
          ox_ref, ot_ref, px_ref, pt_ref, aux_ref,
          wb_ref, c2_ref, w1g_ref, b1g_ref, s_ref, w2_ref,
          imp_ref, load_ref):
    i = pl.program_id(0)
    nsteps = pl.num_programs(0)
    ti = ti_ref[0, 0]

    @pl.when(i == 0)
    def _prep():
        wb_ref[...] = (wr_ref[...] * g1_ref[...]).astype(jnp.bfloat16)
        c2_ref[...] = jax.lax.dot_general(
            b1_ref[...], wr_ref[...], (((1,), (1,)), ((), ())),
            preferred_element_type=jnp.float32)
        we1f = jnp.concatenate([we1_ref[e] for e in range(E)], axis=1)
        wg = wg_ref[ti]
        g2c = g2_ref[...]
        w1g_ref[...] = (jnp.concatenate([we1f, wg], axis=1)
                        * g2c).astype(jnp.bfloat16)
        b1g_ref[...] = jnp.concatenate(
            [jnp.dot(b2_ref[...], we1f, preferred_element_type=jnp.float32)
             + be1_ref[...],
             jnp.dot(b2_ref[...], wg, preferred_element_type=jnp.float32)],
            axis=1)
        col = jax.lax.broadcasted_iota(jnp.int32, (E, EH), 1) // HID
        row = jax.lax.broadcasted_iota(jnp.int32, (E, EH), 0)
        s_ref[...] = (col == row).astype(jnp.bfloat16)
        w2_ref[...] = we2_ref[...].astype(jnp.bfloat16)

    xb = x_ref[...]
    tb = t_ref[...]

    m = (jnp.sum(xb, axis=1, keepdims=True)
         + jnp.sum(tb, axis=1, keepdims=True)) * (1.0 / (2 * DIM))
    v = (jnp.sum(xb * xb, axis=1, keepdims=True)
         + jnp.sum(tb * tb, axis=1, keepdims=True)) * (1.0 / (2 * DIM)) - m * m
    rs = 1.0 / jnp.sqrt(v + 1e-5)

    yn = jnp.concatenate([((xb - m) * rs).astype(jnp.bfloat16),
                          ((tb - m) * rs).astype(jnp.bfloat16)], axis=1)
    u = jax.lax.dot_general(yn, wb_ref[...], (((1,), (1,)), ((), ())),
                            preferred_element_type=jnp.float32)
    u = u + c2_ref[...]

    m2 = jnp.mean(u, axis=1, keepdims=True)
    uc = u - m2
    v2 = jnp.mean(uc * uc, axis=1, keepdims=True)
    zb = (uc * (1.0 / jnp.sqrt(v2 + 1e-5))).astype(jnp.bfloat16)

    r = jnp.dot(zb, w1g_ref[...], preferred_element_type=jnp.float32) + b1g_ref[...]
    h = jnp.maximum(r[:, :EH], 0.0)
    logits = r[:, EH:EH + E]

    pri = (E - jax.lax.broadcasted_iota(jnp.int32, logits.shape, 1)
           ).astype(jnp.float32)
    m1 = jnp.max(logits, axis=1, keepdims=True)
    w1m = jnp.where(logits == m1, pri, 0.0)
    mask1 = w1m == jnp.max(w1m, axis=1, keepdims=True)
    l2 = jnp.where(mask1, -jnp.inf, logits)
    m2g = jnp.max(l2, axis=1, keepdims=True)
    w2m = jnp.where(l2 == m2g, pri, 0.0)
    mask2 = w2m == jnp.max(w2m, axis=1, keepdims=True)
    e2 = jnp.exp(m2g - m1)
    den = 1.0 + e2
    gates = (jnp.where(mask1, 1.0 / den, 0.0)
             + jnp.where(mask2, e2 / den, 0.0))

    gb = gates.astype(jnp.bfloat16)
    gexp = jnp.dot(gb, s_ref[...], preferred_element_type=jnp.float32)
    ghb = (h * gexp).astype(jnp.bfloat16)
    moe = (jnp.dot(ghb, w2_ref[...], preferred_element_type=jnp.float32)
           + jnp.dot(gb, be2_ref[...].astype(jnp.bfloat16),
                     preferred_element_type=jnp.float32))

    px = jax.nn.sigmoid(jnp.mean(moe[:, :RED // 2], axis=1, keepdims=True))
    pt = jax.nn.sigmoid(jnp.mean(moe[:, RED // 2:], axis=1, keepdims=True))

    ox_ref[...] = px * xb + ms_ref[pl.ds(2 * ti, 1), :]
    ot_ref[...] = pt * tb + ms_ref[pl.ds(2 * ti + 1, 1), :]
    px_ref[...] = px
    pt_ref[...] = pt

    imp_b = jnp.sum(gates, axis=0, keepdims=True)
    load_b = jnp.sum((gates > 0.0).astype(jnp.float32), axis=0, keepdims=True)

    @pl.when(i == 0)
    def _init():
        imp_ref[...] = imp_b
        load_ref[...] = load_b

    @pl.when(i > 0)
    def _acc():
        imp_ref[...] += imp_b
        load_ref[...] += load_b

    @pl.when(i == nsteps - 1)
    def _fin():
        def cv2(a):
            mu = jnp.sum(a, axis=1, keepdims=True) * (1.0 / E)
            var = jnp.sum((a - mu) ** 2, axis=1, keepdims=True) * (1.0 / (E - 1))
            return var / (mu * mu + 1e-10)

        aux_ref[...] = (cv2(imp_ref[...]) + cv2(load_ref[...])) * 1e-2


def kernel(x, t, gamma1, beta1, W_red, gamma2, beta2, w_gate, We1, be1, We2,
           be2, modal_shifts, task_index):
    B, N, C = x.shape
    T = B * N
    xf = x.reshape(T, C)
    tf = t.reshape(T, C)
    ti = jnp.asarray(task_index, jnp.int32).reshape(1, 1)

    grid = (T // TB,)
    tok = lambda i: (i, 0)
    fix = lambda i: (0, 0)
    fix3 = lambda i: (0, 0, 0)

    out_x, out_t, pxo, pto, aux = pl.pallas_call(
        _body,
        grid=grid,
        in_specs=[
            pl.BlockSpec((TB, C), tok),
            pl.BlockSpec((TB, C), tok),
            pl.BlockSpec((RED, 2 * C), fix),
            pl.BlockSpec((1, 2 * C), fix),
            pl.BlockSpec((1, 2 * C), fix),
            pl.BlockSpec((RED, 1), fix),
            pl.BlockSpec((1, RED), fix),
            pl.BlockSpec((3, RED, E), fix3),
            pl.BlockSpec((E, RED, HID), fix3),
            pl.BlockSpec((1, EH), fix),
            pl.BlockSpec((EH, RED), fix),
            pl.BlockSpec((E, RED), fix),
            pl.BlockSpec((2 * 3, C), fix),
            pl.BlockSpec(memory_space=pltpu.SMEM),
        ],
        out_specs=[
            pl.BlockSpec((TB, C), tok),
            pl.BlockSpec((TB, C), tok),
            pl.BlockSpec((TB, 1), tok),
            pl.BlockSpec((TB, 1), tok),
            pl.BlockSpec((1, 1), fix),
        ],
        out_shape=[
            jax.ShapeDtypeStruct((T, C), jnp.float32),
            jax.ShapeDtypeStruct((T, C), jnp.float32),
            jax.ShapeDtypeStruct((T, 1), jnp.float32),
            jax.ShapeDtypeStruct((T, 1), jnp.float32),
            jax.ShapeDtypeStruct((1, 1), jnp.float32),
        ],
        scratch_shapes=[
            pltpu.VMEM((RED, 2 * C), jnp.bfloat16),
            pltpu.VMEM((1, RED), jnp.float32),
            pltpu.VMEM((RED, EH + E), jnp.bfloat16),
            pltpu.VMEM((1, EH + E), jnp.float32),
            pltpu.VMEM((E, EH), jnp.bfloat16),
            pltpu.VMEM((EH, RED), jnp.bfloat16),
            pltpu.VMEM((1, E), jnp.float32),
            pltpu.VMEM((1, E), jnp.float32),
        ],
        compiler_params=pltpu.CompilerParams(
            dimension_semantics=("arbitrary",),
        ),
    )(xf, tf, W_red, gamma1[None, :], beta1[None, :], gamma2[:, None],
      beta2[None, :], w_gate, We1, be1.reshape(1, EH), We2.reshape(EH, RED),
      be2, modal_shifts, ti)

    return (out_x.reshape(B, N, C), out_t.reshape(B, N, C),
            pxo.reshape(B, N, 1), pto.reshape(B, N, 1),
            aux.reshape(()))


# speedup vs baseline: 3.7872x; 1.2613x over previous
import jax
import jax.numpy as jnp
from jax.experimental import pallas as pl
from jax.experimental.pallas import tpu as pltpu

DIM = 1024
RED = 256
E = 4
HID = 32
EH = E * HID
TB = 1024
NT = 3


def _body(x_ref, t_ref, wr_hbm, g1_hbm, b1_hbm, g2_hbm, b2_hbm, wg_hbm,
          we1_hbm, be1_hbm, we2_hbm, be2_hbm, ms_hbm, ti_ref,
          ox_ref, ot_ref, px_ref, pt_ref, aux_ref,
          wr_raw, wbx_ref, wbt_ref, c2_ref, g2_v, b2_v, wg_v, we1_v, be1_v,
          w1g_ref, b1g_ref, s_ref, w2_ref, be2_v, ms_v, sem,
          imp_ref, load_ref):
    i = pl.program_id(0)
    nsteps = pl.num_programs(0)
    ti = ti_ref[0, 0]

    @pl.when(i == 0)
    def _prep():
        cps = [pltpu.make_async_copy(wr_hbm, wr_raw.at[0:RED, :], sem),
               pltpu.make_async_copy(g1_hbm, wr_raw.at[RED:RED + 1, :], sem),
               pltpu.make_async_copy(b1_hbm, wr_raw.at[RED + 1:RED + 2, :],
                                     sem),
               pltpu.make_async_copy(g2_hbm, g2_v, sem),
               pltpu.make_async_copy(b2_hbm, b2_v, sem),
               pltpu.make_async_copy(wg_hbm, wg_v, sem),
               pltpu.make_async_copy(we1_hbm, we1_v, sem),
               pltpu.make_async_copy(be1_hbm, be1_v, sem),
               pltpu.make_async_copy(we2_hbm, w2_ref, sem),
               pltpu.make_async_copy(be2_hbm, be2_v, sem),
               pltpu.make_async_copy(ms_hbm, ms_v, sem)]
        for cp in cps:
            cp.start()
        for cp in cps:
            cp.wait()

        wr = wr_raw[0:RED, :]
        g1 = wr_raw[RED:RED + 1, :]
        wbx_ref[...] = (wr[:, :DIM] * g1[:, :DIM]).astype(jnp.bfloat16)
        wbt_ref[...] = (wr[:, DIM:] * g1[:, DIM:]).astype(jnp.bfloat16)
        c2_ref[...] = jax.lax.dot_general(
            wr_raw[RED + 1:RED + 2, :], wr, (((1,), (1,)), ((), ())),
            preferred_element_type=jnp.float32)
        we1f = jnp.concatenate([we1_v[e] for e in range(E)], axis=1)
        w1g_ref[...] = jnp.concatenate([we1f, wg_v[ti]],
                                       axis=1).astype(jnp.bfloat16)
        b1g_ref[...] = jnp.concatenate(
            [be1_v[e:e + 1, :] for e in range(E)]
            + [jnp.zeros((1, E), jnp.float32)], axis=1)
        col = jax.lax.broadcasted_iota(jnp.int32, (E, EH), 1) // HID
        row = jax.lax.broadcasted_iota(jnp.int32, (E, EH), 0)
        s_ref[...] = (col == row).astype(jnp.bfloat16)

    xb = x_ref[...]
    tb = t_ref[...]

    m = (jnp.sum(xb, axis=1, keepdims=True)
         + jnp.sum(tb, axis=1, keepdims=True)) * (1.0 / (2 * DIM))
    v = (jnp.sum(xb * xb, axis=1, keepdims=True)
         + jnp.sum(tb * tb, axis=1, keepdims=True)) * (1.0 / (2 * DIM)) - m * m
    rs = 1.0 / jnp.sqrt(v + 1e-5)

    xnb = ((xb - m) * rs).astype(jnp.bfloat16)
    tnb = ((tb - m) * rs).astype(jnp.bfloat16)
    u = (jax.lax.dot_general(xnb, wbx_ref[...], (((1,), (1,)), ((), ())),
                             preferred_element_type=jnp.float32)
         + jax.lax.dot_general(tnb, wbt_ref[...], (((1,), (1,)), ((), ())),
                               preferred_element_type=jnp.float32))
    u = u + c2_ref[...]

    m2 = jnp.mean(u, axis=1, keepdims=True)
    uc = u - m2
    v2 = jnp.mean(uc * uc, axis=1, keepdims=True)
    yf = uc * (1.0 / jnp.sqrt(v2 + 1e-5)) * g2_v[...] + b2_v[...]
    zb = yf.astype(jnp.bfloat16)

    r = jnp.dot(zb, w1g_ref[...], preferred_element_type=jnp.float32) + b1g_ref[...]
    h = jnp.maximum(r[:, :EH], 0.0)
    logits = r[:, EH:EH + E]

    pri = (E - jax.lax.broadcasted_iota(jnp.int32, logits.shape, 1)
           ).astype(jnp.float32)
    m1 = jnp.max(logits, axis=1, keepdims=True)
    w1m = jnp.where(logits == m1, pri, 0.0)
    mask1 = w1m == jnp.max(w1m, axis=1, keepdims=True)
    l2 = jnp.where(mask1, -jnp.inf, logits)
    m2g = jnp.max(l2, axis=1, keepdims=True)
    w2m = jnp.where(l2 == m2g, pri, 0.0)
    mask2 = w2m == jnp.max(w2m, axis=1, keepdims=True)
    e2 = jnp.exp(m2g - m1)
    den = 1.0 + e2
    gates = (jnp.where(mask1, 1.0 / den, 0.0)
             + jnp.where(mask2, e2 / den, 0.0))

    gb = gates.astype(jnp.bfloat16)
    gexp = jnp.dot(gb, s_ref[...], preferred_element_type=jnp.float32)
    ghb = (h * gexp).astype(jnp.bfloat16)
    moe = (jnp.dot(ghb, w2_ref[...].astype(jnp.bfloat16),
                   preferred_element_type=jnp.float32)
           + jnp.dot(gb, be2_v[...].astype(jnp.bfloat16),
                     preferred_element_type=jnp.float32))

    px = jax.nn.sigmoid(jnp.mean(moe[:, :RED // 2], axis=1, keepdims=True))
    pt = jax.nn.sigmoid(jnp.mean(moe[:, RED // 2:], axis=1, keepdims=True))

    ox_ref[...] = px * xb + ms_v[pl.ds(2 * ti, 1), :]
    ot_ref[...] = pt * tb + ms_v[pl.ds(2 * ti + 1, 1), :]
    px_ref[...] = jnp.reshape(px, (1, TB // 128, 128))
    pt_ref[...] = jnp.reshape(pt, (1, TB // 128, 128))

    imp_b = jnp.sum(gates, axis=0, keepdims=True)
    load_b = jnp.sum((gates > 0.0).astype(jnp.float32), axis=0, keepdims=True)

    @pl.when(i == 0)
    def _init():
        imp_ref[...] = imp_b
        load_ref[...] = load_b

    @pl.when(i > 0)
    def _acc():
        imp_ref[...] += imp_b
        load_ref[...] += load_b

    @pl.when(i == nsteps - 1)
    def _fin():
        def cv2(a):
            mu = jnp.sum(a, axis=1, keepdims=True) * (1.0 / E)
            var = jnp.sum((a - mu) ** 2, axis=1, keepdims=True) * (1.0 / (E - 1))
            return var / (mu * mu + 1e-10)

        aux_ref[...] = (cv2(imp_ref[...]) + cv2(load_ref[...])) * 1e-2


def kernel(x, t, gamma1, beta1, W_red, gamma2, beta2, w_gate, We1, be1, We2,
           be2, modal_shifts, task_index):
    B, N, C = x.shape
    T = B * N
    xf = x.reshape(T, C)
    tf = t.reshape(T, C)
    ti = jnp.asarray(task_index, jnp.int32).reshape(1, 1)

    grid = (T // TB,)
    tok = lambda i: (i, 0)
    fix = lambda i: (0, 0)
    anyspec = pl.BlockSpec(memory_space=pltpu.MemorySpace.HBM)

    out_x, out_t, pxo, pto, aux = pl.pallas_call(
        _body,
        grid=grid,
        in_specs=[
            pl.BlockSpec((TB, C), tok),
            pl.BlockSpec((TB, C), tok),
            anyspec,
            anyspec,
            anyspec,
            anyspec,
            anyspec,
            anyspec,
            anyspec,
            anyspec,
            anyspec,
            anyspec,
            anyspec,
            pl.BlockSpec(memory_space=pltpu.SMEM),
        ],
        out_specs=[
            pl.BlockSpec((TB, C), tok),
            pl.BlockSpec((TB, C), tok),
            pl.BlockSpec((1, TB // 128, 128), lambda i: (i, 0, 0)),
            pl.BlockSpec((1, TB // 128, 128), lambda i: (i, 0, 0)),
            pl.BlockSpec((1, 1), fix),
        ],
        out_shape=[
            jax.ShapeDtypeStruct((T, C), jnp.float32),
            jax.ShapeDtypeStruct((T, C), jnp.float32),
            jax.ShapeDtypeStruct((T // TB, TB // 128, 128), jnp.float32),
            jax.ShapeDtypeStruct((T // TB, TB // 128, 128), jnp.float32),
            jax.ShapeDtypeStruct((1, 1), jnp.float32),
        ],
        scratch_shapes=[
            pltpu.VMEM((RED + 2, 2 * DIM), jnp.float32),
            pltpu.VMEM((RED, DIM), jnp.bfloat16),
            pltpu.VMEM((RED, DIM), jnp.bfloat16),
            pltpu.VMEM((1, RED), jnp.float32),
            pltpu.VMEM((1, RED), jnp.float32),
            pltpu.VMEM((1, RED), jnp.float32),
            pltpu.VMEM((NT, RED, E), jnp.float32),
            pltpu.VMEM((E, RED, HID), jnp.float32),
            pltpu.VMEM((E, HID), jnp.float32),
            pltpu.VMEM((RED, EH + E), jnp.bfloat16),
            pltpu.VMEM((1, EH + E), jnp.float32),
            pltpu.VMEM((E, EH), jnp.bfloat16),
            pltpu.VMEM((EH, RED), jnp.float32),
            pltpu.VMEM((E, RED), jnp.float32),
            pltpu.VMEM((2 * NT, DIM), jnp.float32),
            pltpu.SemaphoreType.DMA,
            pltpu.VMEM((1, E), jnp.float32),
            pltpu.VMEM((1, E), jnp.float32),
        ],
        compiler_params=pltpu.CompilerParams(
            dimension_semantics=("arbitrary",),
        ),
    )(xf, tf, W_red, gamma1[None, :], beta1[None, :], gamma2[None, :],
      beta2[None, :], w_gate, We1, be1, We2.reshape(EH, RED), be2,
      modal_shifts, ti)

    return (out_x.reshape(B, N, C), out_t.reshape(B, N, C),
            pxo.reshape(B, N, 1), pto.reshape(B, N, 1),
            aux.reshape(()))
